# baseline (speedup 1.0000x reference)
import jax
import jax.numpy as jnp
from jax import lax
from jax.experimental import pallas as pl
from jax.experimental.pallas import tpu as pltpu
from jax.experimental.pallas import tpu_sc as plsc

F = 26
D = 64
B = 4096
P = F * (F - 1) // 2
NC, NS = 2, 16
NW = NC * NS
BPW = B // NW
BLK = 16
NBLK = BPW // BLK
FD = F * D
NCHUNK = (P + BLK - 1) // BLK
PREM = P - (NCHUNK - 1) * BLK

_TILES = [(0, 4), (4, 4), (8, 4), (12, 4), (16, 4), (20, 6)]


def _pidx(r, c):
    return r * (2 * F - r - 1) // 2 + (c - r - 1)


def _body(x_hbm, out_hbm, xb0, xb1, xt, oacc, obuf, sem0, sem1):
    cid = lax.axis_index("c")
    sid = lax.axis_index("s")
    wid = sid * NC + cid
    lanes = lax.broadcasted_iota(jnp.int32, (BLK,), 0)
    bscale = lanes * FD

    def start(buf, sem, blk):
        b0 = jnp.minimum(wid * BPW + blk * BLK, B - BLK)
        pltpu.async_copy(x_hbm.at[pl.ds(b0 * FD, BLK * FD)], buf, sem)

    def wait(buf, sem):
        pltpu.make_async_copy(x_hbm.at[pl.ds(0, BLK * FD)], buf, sem).wait()

    def compute(xb, blk):
        def tr_body(j, carry):
            v = plsc.load_gather(xb, [bscale + j])
            xt[j - 4] = carry[0]
            return (carry[1], carry[2], carry[3], v)

        pipe = tuple(plsc.load_gather(xb, [bscale + j]) for j in range(4))
        pipe = lax.fori_loop(4, FD, tr_body, pipe, unroll=8)
        for t in range(4):
            xt[FD - 4 + t] = pipe[t]

        off = blk * BLK
        for ti in range(len(_TILES)):
            r0, rn = _TILES[ti]
            for tj in range(ti, len(_TILES)):
                c0, cn = _TILES[tj]
                pairs = [(u, v) for u in range(rn) for v in range(cn)
                         if (r0 + u) < (c0 + v)]

                def d_body(d, accs, r0=r0, rn=rn, c0=c0, cn=cn,
                           diag=(ti == tj), pairs=pairs):
                    avec = [xt[(r0 + u) * D + d] for u in range(rn)]
                    bvec = avec if diag else [xt[(c0 + v) * D + d]
                                              for v in range(cn)]
                    return tuple(acc + avec[u] * bvec[v]
                                 for acc, (u, v) in zip(accs, pairs))

                init = tuple(jnp.zeros((BLK,), jnp.float32) for _ in pairs)
                accs = lax.fori_loop(0, D, d_body, init, unroll=2)
                for acc, (u, v) in zip(accs, pairs):
                    oacc[pl.ds(_pidx(r0 + u, c0 + v) * BPW + off, BLK)] = acc

    start(xb0, sem0, 0)

    def pair_body(k, _):
        blk0 = k * 2
        wait(xb0, sem0)
        start(xb1, sem1, blk0 + 1)
        compute(xb0, blk0)
        wait(xb1, sem1)
        start(xb0, sem0, blk0 + 2)
        compute(xb1, blk0 + 1)
        return ()

    lax.fori_loop(0, NBLK // 2, pair_body, ())
    wait(xb0, sem0)

    psrc = [(jnp.arange(BLK, dtype=jnp.int32) + p0) * BPW
            for p0 in range(0, P, BLK)]
    last_mask = lanes < PREM
    clamped = jnp.minimum(lanes, PREM - 1) + (NCHUNK - 1) * BLK
    psrc[NCHUNK - 1] = clamped * BPW
    last_cols = clamped

    def row_body(col16, _, g_off):
        col = g_off + col16
        for ch in range(NCHUNK - 1):
            v = plsc.load_gather(oacc, [psrc[ch] + col])
            obuf[col16, pl.ds(ch * BLK, BLK)] = v
        v = plsc.load_gather(oacc, [psrc[NCHUNK - 1] + col])
        plsc.store_scatter(obuf,
                           [jnp.full((BLK,), col16, jnp.int32), last_cols],
                           v, mask=last_mask)
        return ()

    def out_body(g, _):
        lax.fori_loop(0, BLK, lambda c, s: row_body(c, s, g * BLK), ())
        pltpu.sync_copy(obuf, out_hbm.at[pl.ds(wid * BPW + g * BLK, BLK)])
        return ()

    lax.fori_loop(0, NBLK, out_body, ())


def kernel(x):
    xf = x.reshape(B * F * D)
    mesh = plsc.VectorSubcoreMesh(core_axis_name="c", subcore_axis_name="s",
                                  num_cores=NC, num_subcores=NS)
    k = pl.kernel(
        _body,
        out_type=jax.ShapeDtypeStruct((B, P), jnp.float32),
        mesh=mesh,
        compiler_params=pltpu.CompilerParams(needs_layout_passes=False,
                                             use_tc_tiling_on_sc=False),
        scratch_types=[
            pltpu.VMEM((BLK * FD,), jnp.float32),
            pltpu.VMEM((BLK * FD,), jnp.float32),
            pltpu.VMEM((FD, BLK), jnp.float32),
            pltpu.VMEM((P * BPW,), jnp.float32),
            pltpu.VMEM((BLK, P), jnp.float32),
            pltpu.SemaphoreType.DMA,
            pltpu.SemaphoreType.DMA,
        ],
    )
    return k(xf)


# speedup vs baseline: 1.1668x; 1.1668x over previous
import jax
import jax.numpy as jnp
from jax import lax
from jax.experimental import pallas as pl
from jax.experimental.pallas import tpu as pltpu
from jax.experimental.pallas import tpu_sc as plsc

F = 26
D = 64
B = 4096
P = F * (F - 1) // 2
NC, NS = 2, 16
NW = NC * NS
BPW = B // NW
BLK = 16
NBLK = BPW // BLK
FD = F * D
PIPE = 4

_TILES = [(0, 4), (4, 4), (8, 4), (12, 4), (16, 4), (20, 6)]


def _pidx(r, c):
    return r * (2 * F - r - 1) // 2 + (c - r - 1)


def _body(x_hbm, out_hbm, xb0, xb1, xt, oacc, sem0, sem1):
    cid = lax.axis_index("c")
    sid = lax.axis_index("s")
    wid = sid * NC + cid
    lanes = lax.broadcasted_iota(jnp.int32, (BLK,), 0)
    bscale = lanes * FD

    def start(buf, sem, blk):
        b0 = jnp.minimum(wid * BPW + blk * BLK, B - BLK)
        pltpu.async_copy(x_hbm.at[pl.ds(b0 * FD, BLK * FD)], buf, sem)

    def wait(buf, sem):
        pltpu.make_async_copy(x_hbm.at[pl.ds(0, BLK * FD)], buf, sem).wait()

    def compute(xb, blk):
        def tr_body(j, carry):
            v = plsc.load_gather(xb, [bscale + j])
            xt[j - PIPE] = carry[0]
            return carry[1:] + (v,)

        pipe = tuple(plsc.load_gather(xb, [bscale + j]) for j in range(PIPE))
        pipe = lax.fori_loop(PIPE, FD, tr_body, pipe, unroll=8)
        for t in range(PIPE):
            xt[FD - PIPE + t] = pipe[t]

        off = blk * BLK
        for ti in range(len(_TILES)):
            r0, rn = _TILES[ti]
            for tj in range(ti, len(_TILES)):
                c0, cn = _TILES[tj]
                pairs = [(u, v) for u in range(rn) for v in range(cn)
                         if (r0 + u) < (c0 + v)]

                def d_body(d, accs, r0=r0, rn=rn, c0=c0, cn=cn,
                           diag=(ti == tj), pairs=pairs):
                    avec = [xt[(r0 + u) * D + d] for u in range(rn)]
                    bvec = avec if diag else [xt[(c0 + v) * D + d]
                                              for v in range(cn)]
                    return tuple(acc + avec[u] * bvec[v]
                                 for acc, (u, v) in zip(accs, pairs))

                init = tuple(jnp.zeros((BLK,), jnp.float32) for _ in pairs)
                accs = lax.fori_loop(0, D, d_body, init, unroll=2)
                for acc, (u, v) in zip(accs, pairs):
                    oacc[_pidx(r0 + u, c0 + v), pl.ds(off, BLK)] = acc

    start(xb0, sem0, 0)

    def pair_body(k, _):
        blk0 = k * 2
        wait(xb0, sem0)
        start(xb1, sem1, blk0 + 1)
        compute(xb0, blk0)
        wait(xb1, sem1)
        start(xb0, sem0, blk0 + 2)
        compute(xb1, blk0 + 1)
        return ()

    lax.fori_loop(0, NBLK // 2, pair_body, ())
    wait(xb0, sem0)
    pltpu.sync_copy(oacc, out_hbm.at[:, pl.ds(wid * BPW, BPW)])


def kernel(x):
    xf = x.reshape(B * F * D)
    mesh = plsc.VectorSubcoreMesh(core_axis_name="c", subcore_axis_name="s",
                                  num_cores=NC, num_subcores=NS)
    k = pl.kernel(
        _body,
        out_type=jax.ShapeDtypeStruct((P, B), jnp.float32),
        mesh=mesh,
        compiler_params=pltpu.CompilerParams(needs_layout_passes=False,
                                             use_tc_tiling_on_sc=False),
        scratch_types=[
            pltpu.VMEM((BLK * FD,), jnp.float32),
            pltpu.VMEM((BLK * FD,), jnp.float32),
            pltpu.VMEM((FD, BLK), jnp.float32),
            pltpu.VMEM((P, BPW), jnp.float32),
            pltpu.SemaphoreType.DMA,
            pltpu.SemaphoreType.DMA,
        ],
    )
    return k(xf).T
